# baseline (speedup 1.0000x reference)
import functools

import jax
import jax.numpy as jnp
from jax import lax
from jax.experimental import pallas as pl
from jax.experimental.pallas import tpu as pltpu
from jax.experimental.pallas import tpu_sc as plsc

_D = 64
_H = 200
_B = 4096
_OUT = 128
_V = 1000000
_NC = 2
_NS = 16
_NW = _NC * _NS
_BPW = _B // _NW
_S0, _S1 = 104, 96
_NBUF = 4
_UNROLL = 8

_TC_FULL = 999936 // 128
_VTAIL = _V - 999936
_TPB = 2


def _tpose_body(tt_hbm, tail_hbm, out_hbm, ibuf, obuf, tbuf, isem, osem):
    wid = lax.axis_index("s") * _NC + lax.axis_index("c")

    def issue_in(ct, slot):
        for dt in range(8):
            pltpu.async_copy(
                tt_hbm.at[pl.ds(8 * dt, 8), pl.ds(128 * ct, 128)],
                ibuf.at[slot, dt],
                isem,
            )

    def wait_in(slot):
        for dt in range(8):
            pltpu.make_async_copy(
                tt_hbm.at[pl.ds(0, 8), pl.ds(0, 128)],
                ibuf.at[slot, dt],
                isem,
            ).wait()

    def issue_out(ct, slot):
        for ot in range(8):
            pltpu.async_copy(
                obuf.at[slot, ot],
                out_hbm.at[pl.ds(64 * ct + 8 * ot, 8)],
                osem,
            )

    def wait_out(slot):
        for ot in range(8):
            pltpu.make_async_copy(
                obuf.at[slot, ot],
                out_hbm.at[pl.ds(0, 8)],
                osem,
            ).wait()

    lanes = lax.iota(jnp.int32, 16)
    sub = lanes >> 3
    row = lanes & 7

    def transpose_block(slot):
        def obody(o, carry):
            j0 = jnp.full((16,), 2 * o, jnp.int32)
            j1 = j0 + 1
            for cc in range(4):
                dt = sub + 2 * cc
                v = plsc.load_gather(ibuf.at[slot], [dt, row, j0])
                obuf[slot, o >> 3, o & 7, pl.ds(16 * cc, 16)] = v
            for cc in range(4):
                dt = sub + 2 * cc
                v = plsc.load_gather(ibuf.at[slot], [dt, row, j1])
                obuf[slot, o >> 3, o & 7, pl.ds(64 + 16 * cc, 16)] = v
            return carry

        lax.fori_loop(0, 64, obody, 0)

    nblk = (_TC_FULL - 1 - wid) // _NW + 1

    issue_in(wid, 0)

    def blk_body(k, carry):
        slot = k & (_TPB - 1)
        ct = wid + k * _NW
        wait_in(slot)

        @pl.when(k + 1 < nblk)
        def _():
            issue_in(ct + _NW, slot ^ 1)

        @pl.when(k >= _TPB)
        def _():
            wait_out(slot)

        transpose_block(slot)
        issue_out(ct, slot)
        return carry

    lax.fori_loop(0, nblk, blk_body, 0)

    @pl.when(nblk >= 1)
    def _():
        wait_out((nblk - 1) & (_TPB - 1))

    @pl.when(nblk >= 2)
    def _():
        wait_out(nblk & (_TPB - 1))

    @pl.when(wid == 0)
    def _():
        pltpu.sync_copy(tail_hbm, tbuf)
        pltpu.sync_copy(tbuf, out_hbm.at[pl.ds(_V // 2 - 32, 32)])


def _make_tpose():
    mesh = plsc.VectorSubcoreMesh(core_axis_name="c", subcore_axis_name="s")
    return functools.partial(
        pl.kernel,
        mesh=mesh,
        out_type=jax.ShapeDtypeStruct((_V // 2, 128), jnp.float32),
        scratch_types=[
            pltpu.VMEM((_TPB, 8, 8, 128), jnp.float32),
            pltpu.VMEM((_TPB, 8, 8, 128), jnp.float32),
            pltpu.VMEM((32, 128), jnp.float32),
            pltpu.SemaphoreType.DMA,
            pltpu.SemaphoreType.DMA,
        ],
        compiler_params=pltpu.CompilerParams(needs_layout_passes=False),
    )(_tpose_body)


_TPOSE = _make_tpose()


def _pool_body(idx_hbm, table_hbm, out_hbm, idx_v, rows_v, out_v, sem):
    wid = lax.axis_index("s") * _NC + lax.axis_index("c")
    base = wid * _BPW
    pltpu.sync_copy(idx_hbm.at[pl.ds(base, _BPW)], idx_v)

    def issue(r, slot):
        pltpu.async_copy(
            table_hbm.at[idx_v.at[r, pl.ds(0, _S0)]],
            rows_v.at[slot, pl.ds(0, _S0)],
            sem,
        )
        pltpu.async_copy(
            table_hbm.at[idx_v.at[r, pl.ds(_S0, _S1)]],
            rows_v.at[slot, pl.ds(_S0, _S1)],
            sem,
        )

    def wait(slot):
        pltpu.make_async_copy(
            table_hbm.at[idx_v.at[0, pl.ds(0, _S0)]],
            rows_v.at[slot, pl.ds(0, _S0)],
            sem,
        ).wait()
        pltpu.make_async_copy(
            table_hbm.at[idx_v.at[0, pl.ds(_S0, _S1)]],
            rows_v.at[slot, pl.ds(_S0, _S1)],
            sem,
        ).wait()

    def reduce_into(slot, r):
        zero = jnp.zeros((16,), jnp.float32)

        def body(g, accs):
            j = g * _UNROLL
            new = []
            for k in range(_D // 16):
                c = pl.ds(k * 16, 16)
                x = [rows_v[slot, j + u, c] for u in range(_UNROLL)]
                t01 = x[0] + x[1]
                t23 = x[2] + x[3]
                t45 = x[4] + x[5]
                t67 = x[6] + x[7]
                new.append(accs[k] + ((t01 + t23) + (t45 + t67)))
            return tuple(new)

        accs = lax.fori_loop(0, _H // _UNROLL, body, (zero,) * (_D // 16))
        for k in range(_D // 16):
            out_v[r, pl.ds(k * 16, 16)] = accs[k]

    for p in range(_NBUF):
        issue(p, p)

    def outer(g, carry):
        for slot in range(_NBUF):
            r = g * _NBUF + slot
            wait(slot)
            reduce_into(slot, r)
            nxt = r + _NBUF

            @pl.when(nxt < _BPW)
            def _():
                issue(nxt, slot)

        return carry

    lax.fori_loop(0, _BPW // _NBUF, outer, 0)
    pltpu.sync_copy(out_v, out_hbm.at[pl.ds(base, _BPW)])


def _make_pool():
    mesh = plsc.VectorSubcoreMesh(core_axis_name="c", subcore_axis_name="s")
    return functools.partial(
        pl.kernel,
        mesh=mesh,
        out_type=jax.ShapeDtypeStruct((_B, _D), jnp.float32),
        scratch_types=[
            pltpu.VMEM((_BPW, _H), jnp.int32),
            pltpu.VMEM((_NBUF, _H, _D), jnp.float32),
            pltpu.VMEM((_BPW, _D), jnp.float32),
            pltpu.SemaphoreType.DMA,
        ],
        compiler_params=pltpu.CompilerParams(use_tc_tiling_on_sc=False),
    )(_pool_body)


_POOL = _make_pool()


def _linear_body(x_ref, w_ref, b_ref, o_ref):
    x = x_ref[...] * (1.0 / _H)
    o_ref[...] = (
        jnp.dot(x, w_ref[...], preferred_element_type=jnp.float32) + b_ref[...]
    )


_BLK = 1024


def _linear(x, w, b):
    return pl.pallas_call(
        _linear_body,
        grid=(_B // _BLK,),
        in_specs=[
            pl.BlockSpec((_BLK, _D), lambda i: (i, 0)),
            pl.BlockSpec((_D, _OUT), lambda i: (0, 0)),
            pl.BlockSpec((1, _OUT), lambda i: (0, 0)),
        ],
        out_specs=pl.BlockSpec((_BLK, _OUT), lambda i: (i, 0)),
        out_shape=jax.ShapeDtypeStruct((_B, _OUT), jnp.float32),
    )(x, w, b.reshape(1, _OUT))


def kernel(word_indices, embedding, W, b):
    tt = embedding.T
    tail = embedding[999936:].reshape(32, 128)
    packed = _TPOSE(tt, tail)
    table = packed.reshape(_V, _D)
    pooled = _POOL(word_indices.astype(jnp.int32), table)
    return _linear(pooled, W, b)


# speedup vs baseline: 2.6334x; 2.6334x over previous
import functools

import jax
import jax.numpy as jnp
from jax import lax
from jax.experimental import pallas as pl
from jax.experimental.pallas import tpu as pltpu
from jax.experimental.pallas import tpu_sc as plsc

_D = 64
_H = 200
_B = 4096
_OUT = 128
_V = 1000000
_NC = 2
_NS = 16
_NW = _NC * _NS
_BPW = _B // _NW
_S0, _S1 = 104, 96
_NBUF = 4
_UNROLL = 8

_TC_FULL = 999936 // 128
_VTAIL = _V - 999936
_TPB = 2


def _tpose_body(tt_hbm, tail_hbm, out_hbm, ibuf, obuf, tbuf, isem, osem):
    wid = lax.axis_index("s") * _NC + lax.axis_index("c")

    def issue_in(ct, slot):
        for dt in range(8):
            pltpu.async_copy(
                tt_hbm.at[pl.ds(8 * dt, 8), pl.ds(128 * ct, 128)],
                ibuf.at[slot, dt],
                isem,
            )

    def wait_in(slot):
        for dt in range(8):
            pltpu.make_async_copy(
                tt_hbm.at[pl.ds(0, 8), pl.ds(0, 128)],
                ibuf.at[slot, dt],
                isem,
            ).wait()

    def issue_out(ct, slot):
        for ot in range(8):
            pltpu.async_copy(
                obuf.at[slot, ot],
                out_hbm.at[pl.ds(64 * ct + 8 * ot, 8)],
                osem,
            )

    def wait_out(slot):
        for ot in range(8):
            pltpu.make_async_copy(
                obuf.at[slot, ot],
                out_hbm.at[pl.ds(0, 8)],
                osem,
            ).wait()

    lanes = lax.iota(jnp.int32, 16)
    sub = lanes >> 3
    row = lanes & 7

    def transpose_block(slot):
        def sbody(s, carry):
            perm = (lanes + s) & 15
            for a in range(4):
                dt_vec = sub + 2 * a
                for jc in range(8):
                    j_vec = 16 * jc + perm
                    v = plsc.load_gather(
                        ibuf.at[slot], [dt_vec, row, j_vec]
                    )
                    o_vec = j_vec >> 1
                    c_vec = ((j_vec & 1) << 6) + 16 * a + lanes
                    plsc.store_scatter(
                        obuf.at[slot], [o_vec >> 3, o_vec & 7, c_vec], v
                    )
            return carry

        lax.fori_loop(0, 16, sbody, 0)

    nblk = (_TC_FULL - 1 - wid) // _NW + 1

    issue_in(wid, 0)

    def blk_body(k, carry):
        slot = k & (_TPB - 1)
        ct = wid + k * _NW
        wait_in(slot)

        @pl.when(k + 1 < nblk)
        def _():
            issue_in(ct + _NW, slot ^ 1)

        @pl.when(k >= _TPB)
        def _():
            wait_out(slot)

        transpose_block(slot)
        issue_out(ct, slot)
        return carry

    lax.fori_loop(0, nblk, blk_body, 0)

    @pl.when(nblk >= 1)
    def _():
        wait_out((nblk - 1) & (_TPB - 1))

    @pl.when(nblk >= 2)
    def _():
        wait_out(nblk & (_TPB - 1))

    @pl.when(wid == 0)
    def _():
        pltpu.sync_copy(tail_hbm, tbuf)
        pltpu.sync_copy(tbuf, out_hbm.at[pl.ds(_V // 2 - 32, 32)])


def _make_tpose():
    mesh = plsc.VectorSubcoreMesh(core_axis_name="c", subcore_axis_name="s")
    return functools.partial(
        pl.kernel,
        mesh=mesh,
        out_type=jax.ShapeDtypeStruct((_V // 2, 128), jnp.float32),
        scratch_types=[
            pltpu.VMEM((_TPB, 8, 8, 128), jnp.float32),
            pltpu.VMEM((_TPB, 8, 8, 128), jnp.float32),
            pltpu.VMEM((32, 128), jnp.float32),
            pltpu.SemaphoreType.DMA,
            pltpu.SemaphoreType.DMA,
        ],
        compiler_params=pltpu.CompilerParams(needs_layout_passes=False),
    )(_tpose_body)


_TPOSE = _make_tpose()


def _pool_body(idx_hbm, table_hbm, out_hbm, idx_v, rows_v, out_v, sem):
    wid = lax.axis_index("s") * _NC + lax.axis_index("c")
    base = wid * _BPW
    pltpu.sync_copy(idx_hbm.at[pl.ds(base, _BPW)], idx_v)

    def issue(r, slot):
        pltpu.async_copy(
            table_hbm.at[idx_v.at[r, pl.ds(0, _S0)]],
            rows_v.at[slot, pl.ds(0, _S0)],
            sem,
        )
        pltpu.async_copy(
            table_hbm.at[idx_v.at[r, pl.ds(_S0, _S1)]],
            rows_v.at[slot, pl.ds(_S0, _S1)],
            sem,
        )

    def wait(slot):
        pltpu.make_async_copy(
            table_hbm.at[idx_v.at[0, pl.ds(0, _S0)]],
            rows_v.at[slot, pl.ds(0, _S0)],
            sem,
        ).wait()
        pltpu.make_async_copy(
            table_hbm.at[idx_v.at[0, pl.ds(_S0, _S1)]],
            rows_v.at[slot, pl.ds(_S0, _S1)],
            sem,
        ).wait()

    def reduce_into(slot, r):
        zero = jnp.zeros((16,), jnp.float32)

        def body(g, accs):
            j = g * _UNROLL
            new = []
            for k in range(_D // 16):
                c = pl.ds(k * 16, 16)
                x = [rows_v[slot, j + u, c] for u in range(_UNROLL)]
                t01 = x[0] + x[1]
                t23 = x[2] + x[3]
                t45 = x[4] + x[5]
                t67 = x[6] + x[7]
                new.append(accs[k] + ((t01 + t23) + (t45 + t67)))
            return tuple(new)

        accs = lax.fori_loop(0, _H // _UNROLL, body, (zero,) * (_D // 16))
        for k in range(_D // 16):
            out_v[r, pl.ds(k * 16, 16)] = accs[k]

    for p in range(_NBUF):
        issue(p, p)

    def outer(g, carry):
        for slot in range(_NBUF):
            r = g * _NBUF + slot
            wait(slot)
            reduce_into(slot, r)
            nxt = r + _NBUF

            @pl.when(nxt < _BPW)
            def _():
                issue(nxt, slot)

        return carry

    lax.fori_loop(0, _BPW // _NBUF, outer, 0)
    pltpu.sync_copy(out_v, out_hbm.at[pl.ds(base, _BPW)])


def _make_pool():
    mesh = plsc.VectorSubcoreMesh(core_axis_name="c", subcore_axis_name="s")
    return functools.partial(
        pl.kernel,
        mesh=mesh,
        out_type=jax.ShapeDtypeStruct((_B, _D), jnp.float32),
        scratch_types=[
            pltpu.VMEM((_BPW, _H), jnp.int32),
            pltpu.VMEM((_NBUF, _H, _D), jnp.float32),
            pltpu.VMEM((_BPW, _D), jnp.float32),
            pltpu.SemaphoreType.DMA,
        ],
        compiler_params=pltpu.CompilerParams(use_tc_tiling_on_sc=False),
    )(_pool_body)


_POOL = _make_pool()


def _linear_body(x_ref, w_ref, b_ref, o_ref):
    x = x_ref[...] * (1.0 / _H)
    o_ref[...] = (
        jnp.dot(x, w_ref[...], preferred_element_type=jnp.float32) + b_ref[...]
    )


_BLK = 1024


def _linear(x, w, b):
    return pl.pallas_call(
        _linear_body,
        grid=(_B // _BLK,),
        in_specs=[
            pl.BlockSpec((_BLK, _D), lambda i: (i, 0)),
            pl.BlockSpec((_D, _OUT), lambda i: (0, 0)),
            pl.BlockSpec((1, _OUT), lambda i: (0, 0)),
        ],
        out_specs=pl.BlockSpec((_BLK, _OUT), lambda i: (i, 0)),
        out_shape=jax.ShapeDtypeStruct((_B, _OUT), jnp.float32),
    )(x, w, b.reshape(1, _OUT))


def kernel(word_indices, embedding, W, b):
    tt = embedding.T
    tail = embedding[999936:].reshape(32, 128)
    packed = _TPOSE(tt, tail)
    table = packed.reshape(_V, _D)
    pooled = _POOL(word_indices.astype(jnp.int32), table)
    return _linear(pooled, W, b)


# speedup vs baseline: 2.7050x; 1.0272x over previous
import functools

import jax
import jax.numpy as jnp
from jax import lax
from jax.experimental import pallas as pl
from jax.experimental.pallas import tpu as pltpu
from jax.experimental.pallas import tpu_sc as plsc

_D = 64
_H = 200
_B = 4096
_OUT = 128
_V = 1000000
_NC = 2
_NS = 16
_NW = _NC * _NS
_BPW = _B // _NW
_S0, _S1 = 104, 96
_NBUF = 4
_UNROLL = 8

_TC_FULL = 999936 // 128
_VTAIL = _V - 999936
_TPB = 4


def _tpose_body(tt_hbm, tail_hbm, out_hbm, ibuf, obuf, tbuf, isem, osem):
    wid = lax.axis_index("s") * _NC + lax.axis_index("c")

    def issue_in(ct, slot):
        pltpu.async_copy(
            tt_hbm.at[:, :, pl.ds(128 * ct, 128)],
            ibuf.at[slot],
            isem,
        )

    def wait_in(slot):
        pltpu.make_async_copy(
            tt_hbm.at[:, :, pl.ds(0, 128)],
            ibuf.at[slot],
            isem,
        ).wait()

    def issue_out(ct, slot):
        pltpu.async_copy(
            obuf.at[slot],
            out_hbm.at[pl.ds(8 * ct, 8)],
            osem,
        )

    def wait_out(slot):
        pltpu.make_async_copy(
            obuf.at[slot],
            out_hbm.at[pl.ds(0, 8)],
            osem,
        ).wait()

    lanes = lax.iota(jnp.int32, 16)
    sub = lanes >> 3
    row = lanes & 7
    dt_vecs = [sub + 2 * a for a in range(4)]
    base_a = [16 * a + lanes for a in range(4)]

    def transpose_block(slot):
        def sbody(s, carry):
            perm = (lanes + s) & 15
            for jc in range(8):
                j_vec = 16 * jc + perm
                o_vec = j_vec >> 1
                ot = o_vec >> 3
                orr = o_vec & 7
                cpar = (j_vec & 1) << 6
                for a in range(4):
                    v = plsc.load_gather(
                        ibuf.at[slot], [dt_vecs[a], row, j_vec]
                    )
                    plsc.store_scatter(
                        obuf.at[slot], [ot, orr, cpar + base_a[a]], v
                    )
            return carry

        lax.fori_loop(0, 16, sbody, 0)

    nblk = (_TC_FULL - 1 - wid) // _NW + 1

    for p in range(_TPB):
        issue_in(wid + p * _NW, p)

    def blk_body(k, carry):
        slot = k & (_TPB - 1)
        ct = wid + k * _NW
        wait_in(slot)

        @pl.when(k >= _TPB)
        def _():
            wait_out(slot)

        transpose_block(slot)
        issue_out(ct, slot)

        @pl.when(k + _TPB < nblk)
        def _():
            issue_in(ct + _TPB * _NW, slot)

        return carry

    lax.fori_loop(0, nblk, blk_body, 0)

    for p in range(_TPB):
        @pl.when(nblk >= p + 1)
        def _():
            wait_out((nblk - 1 - p) & (_TPB - 1))

    @pl.when(wid == 0)
    def _():
        pltpu.sync_copy(tail_hbm, tbuf)
        pltpu.sync_copy(tbuf, out_hbm.at[pl.ds(62496, 4)])


def _make_tpose():
    mesh = plsc.VectorSubcoreMesh(core_axis_name="c", subcore_axis_name="s")
    return functools.partial(
        pl.kernel,
        mesh=mesh,
        out_type=jax.ShapeDtypeStruct((_V // 16, 8, 128), jnp.float32),
        scratch_types=[
            pltpu.VMEM((_TPB, 8, 8, 128), jnp.float32),
            pltpu.VMEM((_TPB, 8, 8, 128), jnp.float32),
            pltpu.VMEM((4, 8, 128), jnp.float32),
            pltpu.SemaphoreType.DMA,
            pltpu.SemaphoreType.DMA,
        ],
        compiler_params=pltpu.CompilerParams(needs_layout_passes=False),
    )(_tpose_body)


_TPOSE = _make_tpose()


def _pool_body(idx_hbm, table_hbm, out_hbm, idx_v, rows_v, out_v, sem):
    wid = lax.axis_index("s") * _NC + lax.axis_index("c")
    base = wid * _BPW
    pltpu.sync_copy(idx_hbm.at[pl.ds(base, _BPW)], idx_v)

    def issue(r, slot):
        pltpu.async_copy(
            table_hbm.at[idx_v.at[r, pl.ds(0, _S0)]],
            rows_v.at[slot, pl.ds(0, _S0)],
            sem,
        )
        pltpu.async_copy(
            table_hbm.at[idx_v.at[r, pl.ds(_S0, _S1)]],
            rows_v.at[slot, pl.ds(_S0, _S1)],
            sem,
        )

    def wait(slot):
        pltpu.make_async_copy(
            table_hbm.at[idx_v.at[0, pl.ds(0, _S0)]],
            rows_v.at[slot, pl.ds(0, _S0)],
            sem,
        ).wait()
        pltpu.make_async_copy(
            table_hbm.at[idx_v.at[0, pl.ds(_S0, _S1)]],
            rows_v.at[slot, pl.ds(_S0, _S1)],
            sem,
        ).wait()

    def reduce_into(slot, r):
        zero = jnp.zeros((16,), jnp.float32)

        def body(g, accs):
            j = g * _UNROLL
            new = []
            for k in range(_D // 16):
                c = pl.ds(k * 16, 16)
                x = [rows_v[slot, j + u, c] for u in range(_UNROLL)]
                t01 = x[0] + x[1]
                t23 = x[2] + x[3]
                t45 = x[4] + x[5]
                t67 = x[6] + x[7]
                new.append(accs[k] + ((t01 + t23) + (t45 + t67)))
            return tuple(new)

        accs = lax.fori_loop(0, _H // _UNROLL, body, (zero,) * (_D // 16))
        for k in range(_D // 16):
            out_v[r, pl.ds(k * 16, 16)] = accs[k]

    for p in range(_NBUF):
        issue(p, p)

    def outer(g, carry):
        for slot in range(_NBUF):
            r = g * _NBUF + slot
            wait(slot)
            reduce_into(slot, r)
            nxt = r + _NBUF

            @pl.when(nxt < _BPW)
            def _():
                issue(nxt, slot)

        return carry

    lax.fori_loop(0, _BPW // _NBUF, outer, 0)
    pltpu.sync_copy(out_v, out_hbm.at[pl.ds(base, _BPW)])


def _make_pool():
    mesh = plsc.VectorSubcoreMesh(core_axis_name="c", subcore_axis_name="s")
    return functools.partial(
        pl.kernel,
        mesh=mesh,
        out_type=jax.ShapeDtypeStruct((_B, _D), jnp.float32),
        scratch_types=[
            pltpu.VMEM((_BPW, _H), jnp.int32),
            pltpu.VMEM((_NBUF, _H, _D), jnp.float32),
            pltpu.VMEM((_BPW, _D), jnp.float32),
            pltpu.SemaphoreType.DMA,
        ],
        compiler_params=pltpu.CompilerParams(use_tc_tiling_on_sc=False),
    )(_pool_body)


_POOL = _make_pool()


def _linear_body(x_ref, w_ref, b_ref, o_ref):
    x = x_ref[...] * (1.0 / _H)
    o_ref[...] = (
        jnp.dot(x, w_ref[...], preferred_element_type=jnp.float32) + b_ref[...]
    )


_BLK = 1024


def _linear(x, w, b):
    return pl.pallas_call(
        _linear_body,
        grid=(_B // _BLK,),
        in_specs=[
            pl.BlockSpec((_BLK, _D), lambda i: (i, 0)),
            pl.BlockSpec((_D, _OUT), lambda i: (0, 0)),
            pl.BlockSpec((1, _OUT), lambda i: (0, 0)),
        ],
        out_specs=pl.BlockSpec((_BLK, _OUT), lambda i: (i, 0)),
        out_shape=jax.ShapeDtypeStruct((_B, _OUT), jnp.float32),
    )(x, w, b.reshape(1, _OUT))


def kernel(word_indices, embedding, W, b):
    tt = embedding.T.reshape(8, 8, _V)
    tail = embedding[999936:].reshape(4, 8, 128)
    packed = _TPOSE(tt, tail)
    table = packed.reshape(_V, _D)
    pooled = _POOL(word_indices.astype(jnp.int32), table)
    return _linear(pooled, W, b)
